# baseline (speedup 1.0000x reference)
import functools

import jax
import jax.numpy as jnp
from jax.experimental import pallas as pl
from jax.experimental.pallas import tpu as pltpu

B, S, D, V = 128, 16, 2048, 100000
TV = 1000
NSTEP = V // (2 * TV)


def _argmax_body(hs_ref, emb_a, emb_b, bias_a, bias_b, out_ref, best_val,
                 best_idx):
    i = pl.program_id(0)

    def block_minmax(emb, bias, off):
        logits = jax.lax.dot_general(
            hs_ref[...], emb[...], (((1,), (1,)), ((), ())),
            preferred_element_type=jnp.float32,
        ) + bias[0, 0, :][None, :]
        m = jnp.max(logits, axis=1)
        iota = jax.lax.broadcasted_iota(jnp.int32, logits.shape, 1)
        idx = jnp.min(jnp.where(logits == m[:, None], iota, TV), axis=1) + off
        return m, idx

    m_a, idx_a = block_minmax(emb_a, bias_a, i * TV)
    m_b, idx_b = block_minmax(emb_b, bias_b, (V // 2) + i * TV)
    take_b = m_b > m_a
    m = jnp.where(take_b, m_b, m_a)
    idx = jnp.where(take_b, idx_b, idx_a)

    @pl.when(i == 0)
    def _init():
        best_val[...] = m
        best_idx[...] = idx

    @pl.when(i > 0)
    def _update():
        upd = m > best_val[...]
        best_val[...] = jnp.where(upd, m, best_val[...])
        best_idx[...] = jnp.where(upd, idx, best_idx[...])

    @pl.when(i == NSTEP - 1)
    def _emit():
        out_ref[...] = best_idx[...]


@functools.partial(jax.jit, static_argnames=())
def kernel(embedding, hidden_states, output_position, temperatures, top_ps,
           tops_ks, embedding_bias):
    del temperatures, top_ps, tops_ks
    hs = jax.lax.dynamic_slice_in_dim(hidden_states, output_position[0], 1,
                                      axis=1).reshape(B, D)
    bias3d = embedding_bias.reshape(2 * NSTEP, 1, TV)
    out = pl.pallas_call(
        _argmax_body,
        grid=(NSTEP,),
        in_specs=[
            pl.BlockSpec((B, D), lambda i: (0, 0)),
            pl.BlockSpec((TV, D), lambda i: (i, 0)),
            pl.BlockSpec((TV, D), lambda i: (NSTEP + i, 0)),
            pl.BlockSpec((1, 1, TV), lambda i: (i, 0, 0)),
            pl.BlockSpec((1, 1, TV), lambda i: (NSTEP + i, 0, 0)),
        ],
        out_specs=pl.BlockSpec((B,), lambda i: (0,)),
        scratch_shapes=[
            pltpu.VMEM((B,), jnp.float32),
            pltpu.VMEM((B,), jnp.int32),
        ],
        out_shape=jax.ShapeDtypeStruct((B,), jnp.int32),
    )(hs, embedding, embedding, bias3d, bias3d)
    return out


# speedup vs baseline: 1.0198x; 1.0198x over previous
import functools

import jax
import jax.numpy as jnp
from jax.experimental import pallas as pl
from jax.experimental.pallas import tpu as pltpu

B, S, D, V = 128, 16, 2048, 100000
TV = 1000
NSTEP = V // (2 * TV)


def _argmax_body(hs_ref, emb_a, emb_b, bias_a, bias_b, out_ref, best_val,
                 best_idx):
    i = pl.program_id(0)

    def block_minmax(emb, bias, off):
        logits = jax.lax.dot_general(
            hs_ref[...], emb[...], (((1,), (1,)), ((), ())),
            preferred_element_type=jnp.float32,
        ) + bias[0, 0, :][None, :]
        m = jnp.max(logits, axis=1)
        iota = jax.lax.broadcasted_iota(jnp.int32, logits.shape, 1)
        idx = jnp.min(jnp.where(logits == m[:, None], iota, TV), axis=1) + off
        return m, idx

    m_a, idx_a = block_minmax(emb_a, bias_a, i * (2 * TV))
    m_b, idx_b = block_minmax(emb_b, bias_b, i * (2 * TV) + TV)
    take_b = m_b > m_a
    m = jnp.where(take_b, m_b, m_a)
    idx = jnp.where(take_b, idx_b, idx_a)

    @pl.when(i == 0)
    def _init():
        best_val[...] = m
        best_idx[...] = idx

    @pl.when(i > 0)
    def _update():
        upd = m > best_val[...]
        best_val[...] = jnp.where(upd, m, best_val[...])
        best_idx[...] = jnp.where(upd, idx, best_idx[...])

    @pl.when(i == NSTEP - 1)
    def _emit():
        out_ref[...] = best_idx[...]


@functools.partial(jax.jit, static_argnames=())
def kernel(embedding, hidden_states, output_position, temperatures, top_ps,
           tops_ks, embedding_bias):
    del temperatures, top_ps, tops_ks
    hs = jax.lax.dynamic_slice_in_dim(hidden_states, output_position[0], 1,
                                      axis=1).reshape(B, D)
    bias3d = embedding_bias.reshape(2 * NSTEP, 1, TV)
    out = pl.pallas_call(
        _argmax_body,
        grid=(NSTEP,),
        in_specs=[
            pl.BlockSpec((B, D), lambda i: (0, 0)),
            pl.BlockSpec((TV, D), lambda i: (2 * i, 0)),
            pl.BlockSpec((TV, D), lambda i: (2 * i + 1, 0)),
            pl.BlockSpec((1, 1, TV), lambda i: (2 * i, 0, 0)),
            pl.BlockSpec((1, 1, TV), lambda i: (2 * i + 1, 0, 0)),
        ],
        out_specs=pl.BlockSpec((B,), lambda i: (0,)),
        scratch_shapes=[
            pltpu.VMEM((B,), jnp.float32),
            pltpu.VMEM((B,), jnp.int32),
        ],
        out_shape=jax.ShapeDtypeStruct((B,), jnp.int32),
    )(hs, embedding, embedding, bias3d, bias3d)
    return out
